# Initial kernel scaffold; baseline (speedup 1.0000x reference)
import jax
import jax.numpy as jnp
from jax.experimental import pallas as pl


def kernel(z, codebook):
    raise NotImplementedError("write your pallas kernel here")


# speedup vs baseline: 1.1201x; 1.1201x over previous
import jax
import jax.numpy as jnp
from jax.experimental import pallas as pl
from jax.experimental.pallas import tpu as pltpu
from jax.experimental.pallas import tpu_sc as plsc

EMB_DIM = 64
NUM_CODES = 1024
N_ROWS = 16 * 1024
ROW_TILE = 2048
N_TILES = N_ROWS // ROW_TILE
GATHER_WINDOW = 128


def _vq_tc_body(z_ref, cb_ref, zsq_ref, csq_ref, idx_ref, loss_ref):
    z = z_ref[...]
    cb = cb_ref[...]
    dots = jax.lax.dot_general(z, cb, (((1,), (1,)), ((), ())),
                               preferred_element_type=jnp.float32)
    d2 = zsq_ref[...] + csq_ref[...] - 2.0 * dots
    dist = jnp.sqrt(jnp.clip(d2, 0.0, None))
    dmin = jnp.min(dist, axis=1, keepdims=True)
    ids = jax.lax.broadcasted_iota(jnp.int32, dist.shape, 1)
    idx = jnp.min(jnp.where(dist == dmin, ids, NUM_CODES), axis=1,
                  keepdims=True)
    idx_ref[...] = idx
    part = jnp.sum(dmin * dmin, keepdims=True)

    @pl.when(pl.program_id(0) == 0)
    def _init():
        loss_ref[...] = jnp.zeros_like(part)

    loss_ref[...] += part

    @pl.when(pl.program_id(0) == N_TILES - 1)
    def _finish():
        loss_ref[...] = loss_ref[...] * (1.25 / (N_ROWS * EMB_DIM))


def _vq_distances_argmin(z_flat, codebook):
    zsq = jnp.sum(z_flat ** 2, axis=1, keepdims=True)
    csq = jnp.sum(codebook ** 2, axis=1)[None, :]
    return pl.pallas_call(
        _vq_tc_body,
        grid=(N_TILES,),
        in_specs=[
            pl.BlockSpec((ROW_TILE, EMB_DIM), lambda i: (i, 0)),
            pl.BlockSpec((NUM_CODES, EMB_DIM), lambda i: (0, 0)),
            pl.BlockSpec((ROW_TILE, 1), lambda i: (i, 0)),
            pl.BlockSpec((1, NUM_CODES), lambda i: (0, 0)),
        ],
        out_specs=[
            pl.BlockSpec((ROW_TILE, 1), lambda i: (i, 0)),
            pl.BlockSpec((1, 1), lambda i: (0, 0)),
        ],
        out_shape=[
            jax.ShapeDtypeStruct((N_ROWS, 1), jnp.int32),
            jax.ShapeDtypeStruct((1, 1), jnp.float32),
        ],
    )(z_flat, codebook, zsq, csq)


GATHER_WIDTH = 128


def _sc_gather(codebook_padded, indices_2d):
    mesh = plsc.VectorSubcoreMesh(core_axis_name="core",
                                  subcore_axis_name="subcore")

    @pl.kernel(out_type=jax.ShapeDtypeStruct((N_ROWS, GATHER_WIDTH),
                                             jnp.float32),
               mesh=mesh)
    def gather_kernel(cb_hbm, i_hbm, o_hbm):
        def body(i_vmem, o_vmem):
            pltpu.sync_copy(cb_hbm.at[i_vmem.at[0]], o_vmem)

        pltpu.emit_pipeline(
            body,
            grid=(N_ROWS // GATHER_WINDOW,),
            in_specs=[pl.BlockSpec((1, GATHER_WINDOW),
                                   index_map=lambda i: (0, i))],
            out_specs=[pl.BlockSpec((GATHER_WINDOW, GATHER_WIDTH),
                                    index_map=lambda i: (i, 0))],
            core_axis_name=("core", "subcore"),
            dimension_semantics=(pltpu.PARALLEL,),
        )(i_hbm, o_hbm)

    return gather_kernel(codebook_padded, indices_2d)


def kernel(z, codebook):
    z_flat = z.reshape(-1, EMB_DIM)
    idx2d, loss = _vq_distances_argmin(z_flat, codebook)
    cb_padded = jnp.pad(codebook, ((0, 0), (0, GATHER_WIDTH - EMB_DIM)))
    z_q = _sc_gather(cb_padded, idx2d.reshape(1, N_ROWS))
    encoding_indices = idx2d.reshape(N_ROWS)
    return (z_q[:, :EMB_DIM].reshape(z.shape), loss.reshape(()),
            encoding_indices)
